# Initial kernel scaffold; baseline (speedup 1.0000x reference)
import jax
import jax.numpy as jnp
from jax.experimental import pallas as pl


def kernel(paragraph_variable, sentence_length_list, paragh_length_list, max_no_lines, W_emb, W_ih, W_hh, b_ih, b_hh):
    raise NotImplementedError("write your pallas kernel here")


# speedup vs baseline: 25.1897x; 25.1897x over previous
import jax
import jax.numpy as jnp
from jax import lax
from jax.experimental import pallas as pl
from jax.experimental.pallas import tpu as pltpu
from jax.experimental.pallas import tpu_sc as plsc

B = 16
L = 256
T = 2048
D = 256
H = 256

NC = 2
NS = 16
NW = NC * NS
CHUNK = 128
SLOTS = 8
SENT_W = 128
SENT_CHUNK = CHUNK // SLOTS
NCHUNK = SENT_W // SENT_CHUNK

CL = 16
NBLK = L // CL


def _sc_body(tok_hbm, msk_hbm, sidx_hbm, wemb_hbm, out_hbm,
             tok_v, msk_v, sidx_v, rows_v, outbuf_v, sem):
    c = lax.axis_index("c")
    s = lax.axis_index("s")
    w = c * NS + s

    pltpu.sync_copy(tok_hbm.at[w], tok_v)
    pltpu.sync_copy(msk_hbm.at[w], msk_v)
    pltpu.sync_copy(sidx_hbm.at[w], sidx_v.at[0])

    def chunk_body(j, carry):
        pltpu.async_copy(wemb_hbm.at[tok_v.at[j]], rows_v, sem).wait()

        def pair_body(p, inner):
            mv = msk_v[j, pl.ds(p * 16, 16)]
            for half in range(2):
                si = p * 2 + half
                r0 = si * SLOTS
                acc = [jnp.zeros((16,), jnp.float32) for _ in range(D // 16)]
                for k in range(SLOTS):
                    m = mv[half * SLOTS + k]
                    for cg in range(D // 16):
                        acc[cg] = (acc[cg]
                                   + rows_v[r0 + k, pl.ds(cg * 16, 16)] * m)
                orow = j * SENT_CHUNK + si
                for cg in range(D // 16):
                    outbuf_v[orow, pl.ds(cg * 16, 16)] = acc[cg]
            return inner

        return lax.fori_loop(0, SENT_CHUNK // 2, pair_body, carry)

    lax.fori_loop(0, NCHUNK, chunk_body, 0)

    pltpu.async_copy(outbuf_v, out_hbm.at[sidx_v.at[0]], sem).wait()


def _segment_sum_sc(tok, msk, sidx, W_emb):
    mesh = plsc.VectorSubcoreMesh(core_axis_name="c", subcore_axis_name="s")
    fn = pl.kernel(
        _sc_body,
        out_type=jax.ShapeDtypeStruct((L * B, D), jnp.float32),
        mesh=mesh,
        scratch_types=[
            pltpu.VMEM((NCHUNK, CHUNK), jnp.int32),
            pltpu.VMEM((NCHUNK, CHUNK), jnp.float32),
            pltpu.VMEM((1, SENT_W), jnp.int32),
            pltpu.VMEM((CHUNK, D), jnp.float32),
            pltpu.VMEM((SENT_W, D), jnp.float32),
            pltpu.SemaphoreType.DMA,
        ],
    )
    return fn(tok, msk, sidx, W_emb)


def _lstm_body(x_ref, wih_ref, whh_ref, bias_ref, len_ref, out_ref,
               h_ref, c_ref, gx_ref):
    blk = pl.program_id(0)

    @pl.when(blk == 0)
    def _():
        h_ref[...] = jnp.zeros_like(h_ref)
        c_ref[...] = jnp.zeros_like(c_ref)

    x2 = x_ref[...].reshape(CL * B, D)
    gx_ref[...] = (
        jnp.dot(x2, wih_ref[...], preferred_element_type=jnp.float32)
        + bias_ref[...]
    )

    lens = len_ref[...][:, 0:1]

    def step(j, carry):
        h, c = carry
        t = blk * CL + j
        gates = gx_ref[pl.ds(j * B, B)] + jnp.dot(
            h, whh_ref[...], preferred_element_type=jnp.float32)
        ii = jax.nn.sigmoid(gates[:, 0:H])
        ff = jax.nn.sigmoid(gates[:, H:2 * H])
        gg = jnp.tanh(gates[:, 2 * H:3 * H])
        oo = jax.nn.sigmoid(gates[:, 3 * H:4 * H])
        c_new = ff * c + ii * gg
        h_new = oo * jnp.tanh(c_new)
        mask = t < lens
        out_ref[j] = jnp.where(mask, h_new, 0.0)
        return (jnp.where(mask, h_new, h), jnp.where(mask, c_new, c))

    hh, cc = lax.fori_loop(0, CL, step, (h_ref[...], c_ref[...]))
    h_ref[...] = hh
    c_ref[...] = cc


def _lstm_tc(x, wihT, whhT, bias, lens):
    return pl.pallas_call(
        _lstm_body,
        grid=(NBLK,),
        in_specs=[
            pl.BlockSpec((CL, B, D), lambda i: (i, 0, 0)),
            pl.BlockSpec((D, 4 * H), lambda i: (0, 0)),
            pl.BlockSpec((H, 4 * H), lambda i: (0, 0)),
            pl.BlockSpec((1, 4 * H), lambda i: (0, 0)),
            pl.BlockSpec((B, 128), lambda i: (0, 0)),
        ],
        out_specs=pl.BlockSpec((CL, B, H), lambda i: (i, 0, 0)),
        out_shape=jax.ShapeDtypeStruct((L, B, H), jnp.float32),
        scratch_shapes=[
            pltpu.VMEM((B, H), jnp.float32),
            pltpu.VMEM((B, H), jnp.float32),
            pltpu.VMEM((CL * B, 4 * H), jnp.float32),
        ],
    )(x, wihT, whhT, bias, lens)


def kernel(paragraph_variable, sentence_length_list, paragh_length_list,
           max_no_lines, W_emb, W_ih, W_hh, b_ih, b_hh):
    sll = sentence_length_list.astype(jnp.int32)
    ends = jnp.cumsum(sll, axis=1)
    starts = ends - sll

    h_arr = jnp.arange(NW, dtype=jnp.int32) % 2
    b_of_w = jnp.arange(NW, dtype=jnp.int32) // 2
    g_idx = h_arr[:, None] * SENT_W + jnp.arange(SENT_W, dtype=jnp.int32)
    st = starts[b_of_w[:, None], g_idx]
    ln = sll[b_of_w[:, None], g_idx]

    k_arr = jnp.arange(SLOTS, dtype=jnp.int32)
    pos = jnp.minimum(st[:, :, None] + k_arr, T - 1)
    tok = paragraph_variable.astype(jnp.int32)[b_of_w[:, None, None], pos]
    msk = (k_arr[None, None, :] < ln[:, :, None]).astype(jnp.float32)

    tok = tok.reshape(NW, NCHUNK, CHUNK)
    msk = msk.reshape(NW, NCHUNK, CHUNK)

    j_arr = jnp.arange(SENT_W, dtype=jnp.int32)
    sidx = (h_arr[:, None] * SENT_W + j_arr[None, :]) * B + b_of_w[:, None]

    lineflat = _segment_sum_sc(tok, msk, sidx, W_emb)
    x = lineflat.reshape(L, B, D)

    wihT = W_ih.T
    whhT = W_hh.T
    bias = (b_ih + b_hh).reshape(1, 4 * H)
    lens = jnp.broadcast_to(
        paragh_length_list.astype(jnp.int32)[:, None], (B, 128))

    return _lstm_tc(x, wihT, whhT, bias, lens)


# speedup vs baseline: 28.8999x; 1.1473x over previous
import functools

import jax
import jax.numpy as jnp
from jax import lax
from jax.experimental import pallas as pl
from jax.experimental.pallas import tpu as pltpu
from jax.experimental.pallas import tpu_sc as plsc

B = 16
L = 256
T = 2048
D = 256
H = 256

NC = 2
NS = 16
NW = NC * NS
CHUNK = 128
SLOTS = 8
LH = L // 2
SENTS = LH // 2
SENT_CHUNK = CHUNK // SLOTS
NCHUNK = SENTS // SENT_CHUNK

CL = 16
NBLK = LH // CL


def _accum_chunk(j, rows_v, msk_v, outbuf_v):
    def pair_body(p, inner):
        mv = msk_v[j, pl.ds(p * 16, 16)]
        for half in range(2):
            si = p * 2 + half
            r0 = si * SLOTS
            acc = [jnp.zeros((16,), jnp.float32) for _ in range(D // 16)]
            for k in range(SLOTS):
                m = mv[half * SLOTS + k]
                for cg in range(D // 16):
                    acc[cg] = (acc[cg]
                               + rows_v[r0 + k, pl.ds(cg * 16, 16)] * m)
            orow = j * SENT_CHUNK + si
            for cg in range(D // 16):
                outbuf_v[orow, pl.ds(cg * 16, 16)] = acc[cg]
        return inner

    return lax.fori_loop(0, SENT_CHUNK // 2, pair_body, 0)


def _sc_body(tok_hbm, msk_hbm, sidx_hbm, wemb_hbm, out_hbm,
             tokids_v, msk_v, sidx_v, rows0_v, rows1_v, outbuf_v,
             sem0, sem1):
    c = lax.axis_index("c")
    s = lax.axis_index("s")
    w = c * NS + s

    pltpu.sync_copy(tok_hbm.at[w], tokids_v)
    pltpu.sync_copy(msk_hbm.at[w], msk_v)
    pltpu.sync_copy(sidx_hbm.at[w], sidx_v.at[0])

    pltpu.async_copy(wemb_hbm.at[tokids_v.at[0]], rows0_v, sem0)

    def pair_chunks(t, carry):
        j0 = t * 2
        j1 = j0 + 1
        pltpu.make_async_copy(
            wemb_hbm.at[tokids_v.at[j0]], rows0_v, sem0).wait()
        pltpu.async_copy(wemb_hbm.at[tokids_v.at[j1]], rows1_v, sem1)
        _accum_chunk(j0, rows0_v, msk_v, outbuf_v)
        pltpu.make_async_copy(
            wemb_hbm.at[tokids_v.at[j1]], rows1_v, sem1).wait()

        @pl.when(t < NCHUNK // 2 - 1)
        def _():
            pltpu.async_copy(wemb_hbm.at[tokids_v.at[j0 + 2]], rows0_v, sem0)

        _accum_chunk(j1, rows1_v, msk_v, outbuf_v)
        return carry

    lax.fori_loop(0, NCHUNK // 2, pair_chunks, 0)

    pltpu.async_copy(outbuf_v, out_hbm.at[sidx_v.at[0]], sem0).wait()


def _segment_sum_sc(tok, msk, sidx, W_emb):
    mesh = plsc.VectorSubcoreMesh(core_axis_name="c", subcore_axis_name="s")
    fn = pl.kernel(
        _sc_body,
        out_type=jax.ShapeDtypeStruct((LH * B, D), jnp.float32),
        mesh=mesh,
        scratch_types=[
            pltpu.VMEM((NCHUNK, CHUNK), jnp.int32),
            pltpu.VMEM((NCHUNK, CHUNK), jnp.float32),
            pltpu.VMEM((1, SENTS), jnp.int32),
            pltpu.VMEM((CHUNK, D), jnp.float32),
            pltpu.VMEM((CHUNK, D), jnp.float32),
            pltpu.VMEM((SENTS, D), jnp.float32),
            pltpu.SemaphoreType.DMA,
            pltpu.SemaphoreType.DMA,
        ],
    )
    return fn(tok, msk, sidx, W_emb)


def _lstm_body(toff, maxlen_ref, x_ref, wih_ref, whh_ref, bias_ref, len_ref,
               hin_ref, cin_ref, out_ref, hout_ref, cout_ref,
               h_ref, c_ref, gx_ref):
    blk = pl.program_id(0)

    @pl.when(blk == 0)
    def _():
        h_ref[...] = hin_ref[...]
        c_ref[...] = cin_ref[...]

    active = toff + blk * CL < maxlen_ref[0]

    @pl.when(active)
    def _():
        x2 = x_ref[...].reshape(CL * B, D)
        gx_ref[...] = (
            jnp.dot(x2, wih_ref[...], preferred_element_type=jnp.float32)
            + bias_ref[...]
        )

        lens = len_ref[...][:, 0:1]

        def step(j, carry):
            h, c = carry
            t = toff + blk * CL + j
            gates = gx_ref[pl.ds(j * B, B)] + jnp.dot(
                h, whh_ref[...], preferred_element_type=jnp.float32)
            ii = jax.nn.sigmoid(gates[:, 0:H])
            ff = jax.nn.sigmoid(gates[:, H:2 * H])
            gg = jnp.tanh(gates[:, 2 * H:3 * H])
            oo = jax.nn.sigmoid(gates[:, 3 * H:4 * H])
            c_new = ff * c + ii * gg
            h_new = oo * jnp.tanh(c_new)
            mask = t < lens
            out_ref[j] = jnp.where(mask, h_new, 0.0)
            return (jnp.where(mask, h_new, h), jnp.where(mask, c_new, c))

        hh, cc = lax.fori_loop(0, CL, step, (h_ref[...], c_ref[...]))
        h_ref[...] = hh
        c_ref[...] = cc

    @pl.when(jnp.logical_not(active))
    def _():
        out_ref[...] = jnp.zeros_like(out_ref)

    @pl.when(blk == NBLK - 1)
    def _():
        hout_ref[...] = h_ref[...]
        cout_ref[...] = c_ref[...]


def _lstm_tc(toff, maxlen, x, wihT, whhT, bias, lens, h0, c0):
    return pl.pallas_call(
        functools.partial(_lstm_body, toff),
        grid=(NBLK,),
        in_specs=[
            pl.BlockSpec(memory_space=pltpu.SMEM),
            pl.BlockSpec((CL, B, D), lambda i: (i, 0, 0)),
            pl.BlockSpec((D, 4 * H), lambda i: (0, 0)),
            pl.BlockSpec((H, 4 * H), lambda i: (0, 0)),
            pl.BlockSpec((1, 4 * H), lambda i: (0, 0)),
            pl.BlockSpec((B, 128), lambda i: (0, 0)),
            pl.BlockSpec((B, H), lambda i: (0, 0)),
            pl.BlockSpec((B, H), lambda i: (0, 0)),
        ],
        out_specs=[
            pl.BlockSpec((CL, B, H), lambda i: (i, 0, 0)),
            pl.BlockSpec((B, H), lambda i: (0, 0)),
            pl.BlockSpec((B, H), lambda i: (0, 0)),
        ],
        out_shape=[
            jax.ShapeDtypeStruct((LH, B, H), jnp.float32),
            jax.ShapeDtypeStruct((B, H), jnp.float32),
            jax.ShapeDtypeStruct((B, H), jnp.float32),
        ],
        scratch_shapes=[
            pltpu.VMEM((B, H), jnp.float32),
            pltpu.VMEM((B, H), jnp.float32),
            pltpu.VMEM((CL * B, 4 * H), jnp.float32),
        ],
    )(maxlen, x, wihT, whhT, bias, lens, h0, c0)


def kernel(paragraph_variable, sentence_length_list, paragh_length_list,
           max_no_lines, W_emb, W_ih, W_hh, b_ih, b_hh):
    sll = sentence_length_list.astype(jnp.int32)
    ends = jnp.cumsum(sll, axis=1)
    st4 = (ends - sll).reshape(B, 2, 2, SENTS)
    ln4 = sll.reshape(B, 2, 2, SENTS)
    st_w = jnp.transpose(st4, (0, 2, 1, 3)).reshape(NW, 2, SENTS)
    ln_w = jnp.transpose(ln4, (0, 2, 1, 3)).reshape(NW, 2, SENTS)

    k_arr = jnp.arange(SLOTS, dtype=jnp.int32)
    pos = jnp.minimum(st_w[..., None] + k_arr, T - 1)
    msk = (k_arr < ln_w[..., None]).astype(jnp.float32)

    pv_w = jnp.repeat(paragraph_variable.astype(jnp.int32), 2, axis=0)
    tok = jnp.take_along_axis(pv_w, pos.reshape(NW, 2 * SENTS * SLOTS), axis=1)
    tok = tok.reshape(NW, 2, NCHUNK, CHUNK)
    msk = msk.reshape(NW, 2, NCHUNK, CHUNK)

    hh_arr = jnp.arange(NW, dtype=jnp.int32) % 2
    b_of_w = jnp.arange(NW, dtype=jnp.int32) // 2
    j_arr = jnp.arange(SENTS, dtype=jnp.int32)
    sidx = (hh_arr[:, None] * SENTS + j_arr[None, :]) * B + b_of_w[:, None]

    wihT = W_ih.T
    whhT = W_hh.T
    bias = (b_ih + b_hh).reshape(1, 4 * H)
    lens = jnp.broadcast_to(
        paragh_length_list.astype(jnp.int32)[:, None], (B, 128))
    maxlen = paragh_length_list.astype(jnp.int32)[:1]
    zhc = jnp.zeros((B, H), jnp.float32)

    xa = _segment_sum_sc(tok[:, 0], msk[:, 0], sidx, W_emb).reshape(LH, B, D)
    xb = _segment_sum_sc(tok[:, 1], msk[:, 1], sidx, W_emb).reshape(LH, B, D)

    out_a, h1, c1 = _lstm_tc(0, maxlen, xa, wihT, whhT, bias, lens, zhc, zhc)
    out_b, _, _ = _lstm_tc(LH, maxlen, xb, wihT, whhT, bias, lens, h1, c1)

    return jnp.concatenate([out_a, out_b], axis=0)
